# baseline (speedup 1.0000x reference)
import functools

import jax
import jax.numpy as jnp
from jax import lax
from jax.experimental import pallas as pl
from jax.experimental.pallas import tpu as pltpu
from jax.experimental.pallas import tpu_sc as plsc

_CUTOFF = 5.0
_L = 16
_NC = 2
_NS = 16
_NW = _NC * _NS
_CHUNK = 4000

_PI = 3.141592653589793
_A0 = 0.5
_A1 = -0.5 / 6.0
_A2 = 0.5 / 120.0
_A3 = -0.5 / 5040.0
_A4 = 0.5 / 362880.0


@functools.lru_cache(maxsize=None)
def _make_kernel(E_in: int, E_f: int):
    per_w = E_f // _NW
    steps = per_w // _CHUNK
    assert per_w * _NW == E_f and steps * _CHUNK == per_w

    mesh = plsc.VectorSubcoreMesh(
        core_axis_name="c", subcore_axis_name="s",
        num_cores=_NC, num_subcores=_NS)

    @functools.partial(
        pl.kernel,
        out_type=(
            jax.ShapeDtypeStruct((E_f,), jnp.float32),
            jax.ShapeDtypeStruct((E_f,), jnp.float32),
            jax.ShapeDtypeStruct((E_f,), jnp.float32),
            jax.ShapeDtypeStruct((E_f,), jnp.float32),
            jax.ShapeDtypeStruct((E_f,), jnp.float32),
            jax.ShapeDtypeStruct((E_f,), jnp.int32),
        ),
        mesh=mesh,
        scratch_types=[
            pltpu.VMEM((_CHUNK,), jnp.int32),
            pltpu.VMEM((_CHUNK,), jnp.int32),
            pltpu.VMEM((_CHUNK,), jnp.float32),
            pltpu.VMEM((_CHUNK,), jnp.float32),
            pltpu.VMEM((_CHUNK,), jnp.float32),
            pltpu.VMEM((_CHUNK,), jnp.float32),
            pltpu.VMEM((_CHUNK,), jnp.float32),
            pltpu.VMEM((_CHUNK,), jnp.int32),
            pltpu.SemaphoreType.DMA,
        ],
    )
    def k(t0_hbm, t1_hbm, t2_hbm, dist_hbm, idx_hbm,
          p0_hbm, p1_hbm, p2_hbm, distf_hbm, sw_hbm, msk_hbm,
          idx_raw, idx_c, g0, g1, g2, gd, sw_v, msk_v, sem):
        wid = lax.axis_index("s") * _NC + lax.axis_index("c")
        base0 = wid * per_w

        def step(s, carry):
            base = base0 + s * _CHUNK
            pltpu.sync_copy(idx_hbm.at[pl.ds(base, _CHUNK)], idx_raw)

            def clamp(t, c):
                off = t * _L
                iv = idx_raw[pl.ds(off, _L)]
                idx_c[pl.ds(off, _L)] = jnp.minimum(iv, E_in - 1)
                return c
            lax.fori_loop(0, _CHUNK // _L, clamp, 0)

            c0 = pltpu.async_copy(t0_hbm.at[idx_c], g0, sem)
            c1 = pltpu.async_copy(t1_hbm.at[idx_c], g1, sem)
            c2 = pltpu.async_copy(t2_hbm.at[idx_c], g2, sem)
            cd = pltpu.async_copy(dist_hbm.at[idx_c], gd, sem)
            c0.wait()
            c1.wait()
            c2.wait()
            cd.wait()

            def compute(t, c):
                off = t * _L
                iv = idx_raw[pl.ds(off, _L)]
                valid = iv < E_in
                dg = gd[pl.ds(off, _L)]
                df = jnp.where(valid, dg, _CUTOFF)
                mask = df < _CUTOFF
                u = (df * (1.0 / _CUTOFF) - 0.5) * _PI
                u2 = u * u
                p = _A4 * u2 + _A3
                p = p * u2 + _A2
                p = p * u2 + _A1
                p = p * u2 + _A0
                sw = jnp.where(mask, 0.5 - u * p, 0.0)
                gd[pl.ds(off, _L)] = df
                sw_v[pl.ds(off, _L)] = sw
                msk_v[pl.ds(off, _L)] = jnp.where(mask, 1, 0)
                g0[pl.ds(off, _L)] = jnp.where(valid, g0[pl.ds(off, _L)], _CUTOFF)
                g1[pl.ds(off, _L)] = jnp.where(valid, g1[pl.ds(off, _L)], _CUTOFF)
                g2[pl.ds(off, _L)] = jnp.where(valid, g2[pl.ds(off, _L)], _CUTOFF)
                return c
            lax.fori_loop(0, _CHUNK // _L, compute, 0)

            pltpu.sync_copy(g0, p0_hbm.at[pl.ds(base, _CHUNK)])
            pltpu.sync_copy(g1, p1_hbm.at[pl.ds(base, _CHUNK)])
            pltpu.sync_copy(g2, p2_hbm.at[pl.ds(base, _CHUNK)])
            pltpu.sync_copy(gd, distf_hbm.at[pl.ds(base, _CHUNK)])
            pltpu.sync_copy(sw_v, sw_hbm.at[pl.ds(base, _CHUNK)])
            pltpu.sync_copy(msk_v, msk_hbm.at[pl.ds(base, _CHUNK)])
            return carry

        lax.fori_loop(0, steps, step, 0)

    return k


def kernel(vec, distances, filter_indices):
    E_in = vec.shape[0]
    E_f = filter_indices.shape[0]
    p0, p1, p2, dist_f, switch, msk = _make_kernel(E_in, E_f)(
        vec[:, 0], vec[:, 1], vec[:, 2], distances, filter_indices)
    vec_f = jnp.stack([p0, p1, p2], axis=1)
    return vec_f, dist_f, switch, msk.astype(jnp.bool_)


# speedup vs baseline: 1.0320x; 1.0320x over previous
import functools

import jax
import jax.numpy as jnp
from jax import lax
from jax.experimental import pallas as pl
from jax.experimental.pallas import tpu as pltpu
from jax.experimental.pallas import tpu_sc as plsc

_CUTOFF = 5.0
_L = 16
_NC = 2
_NS = 16
_NW = _NC * _NS
_CHUNK = 2000

_PI = 3.141592653589793
_A0 = 0.5
_A1 = -0.5 / 6.0
_A2 = 0.5 / 120.0
_A3 = -0.5 / 5040.0
_A4 = 0.5 / 362880.0


@functools.lru_cache(maxsize=None)
def _make_kernel(E_in: int, E_f: int):
    per_w = E_f // _NW
    steps = per_w // _CHUNK
    assert per_w * _NW == E_f and steps * _CHUNK == per_w

    mesh = plsc.VectorSubcoreMesh(
        core_axis_name="c", subcore_axis_name="s",
        num_cores=_NC, num_subcores=_NS)

    @functools.partial(
        pl.kernel,
        out_type=(
            jax.ShapeDtypeStruct((E_f,), jnp.float32),
            jax.ShapeDtypeStruct((E_f,), jnp.float32),
            jax.ShapeDtypeStruct((E_f,), jnp.float32),
            jax.ShapeDtypeStruct((E_f,), jnp.float32),
            jax.ShapeDtypeStruct((E_f,), jnp.float32),
            jax.ShapeDtypeStruct((E_f,), jnp.int32),
        ),
        mesh=mesh,
        scratch_types=[
            pltpu.VMEM((_CHUNK,), jnp.int32),
            pltpu.VMEM((_CHUNK,), jnp.int32),
            pltpu.VMEM((_CHUNK,), jnp.float32),
            pltpu.VMEM((_CHUNK,), jnp.float32),
            pltpu.VMEM((_CHUNK,), jnp.float32),
            pltpu.VMEM((_CHUNK,), jnp.float32),
            pltpu.VMEM((_CHUNK,), jnp.float32),
            pltpu.VMEM((_CHUNK,), jnp.int32),
            pltpu.SemaphoreType.DMA,
        ],
    )
    def k(t0_hbm, t1_hbm, t2_hbm, dist_hbm, idx_hbm,
          p0_hbm, p1_hbm, p2_hbm, distf_hbm, sw_hbm, msk_hbm,
          idx_raw, idx_c, g0, g1, g2, gd, sw_v, msk_v, sem):
        wid = lax.axis_index("s") * _NC + lax.axis_index("c")
        base0 = wid * per_w

        def step(s, carry):
            base = base0 + s * _CHUNK
            pltpu.sync_copy(idx_hbm.at[pl.ds(base, _CHUNK)], idx_raw)

            def clamp(t, c):
                off = t * _L
                iv = idx_raw[pl.ds(off, _L)]
                idx_c[pl.ds(off, _L)] = jnp.minimum(iv, E_in - 1)
                return c
            lax.fori_loop(0, _CHUNK // _L, clamp, 0)

            c0 = pltpu.async_copy(t0_hbm.at[idx_c], g0, sem)
            c1 = pltpu.async_copy(t1_hbm.at[idx_c], g1, sem)
            c2 = pltpu.async_copy(t2_hbm.at[idx_c], g2, sem)
            cd = pltpu.async_copy(dist_hbm.at[idx_c], gd, sem)
            c0.wait()
            c1.wait()
            c2.wait()
            cd.wait()

            def compute(t, c):
                off = t * _L
                iv = idx_raw[pl.ds(off, _L)]
                valid = iv < E_in
                dg = gd[pl.ds(off, _L)]
                df = jnp.where(valid, dg, _CUTOFF)
                mask = df < _CUTOFF
                u = (df * (1.0 / _CUTOFF) - 0.5) * _PI
                u2 = u * u
                p = _A4 * u2 + _A3
                p = p * u2 + _A2
                p = p * u2 + _A1
                p = p * u2 + _A0
                sw = jnp.where(mask, 0.5 - u * p, 0.0)
                gd[pl.ds(off, _L)] = df
                sw_v[pl.ds(off, _L)] = sw
                msk_v[pl.ds(off, _L)] = jnp.where(mask, 1, 0)
                g0[pl.ds(off, _L)] = jnp.where(valid, g0[pl.ds(off, _L)], _CUTOFF)
                g1[pl.ds(off, _L)] = jnp.where(valid, g1[pl.ds(off, _L)], _CUTOFF)
                g2[pl.ds(off, _L)] = jnp.where(valid, g2[pl.ds(off, _L)], _CUTOFF)
                return c
            lax.fori_loop(0, _CHUNK // _L, compute, 0)

            pltpu.sync_copy(g0, p0_hbm.at[pl.ds(base, _CHUNK)])
            pltpu.sync_copy(g1, p1_hbm.at[pl.ds(base, _CHUNK)])
            pltpu.sync_copy(g2, p2_hbm.at[pl.ds(base, _CHUNK)])
            pltpu.sync_copy(gd, distf_hbm.at[pl.ds(base, _CHUNK)])
            pltpu.sync_copy(sw_v, sw_hbm.at[pl.ds(base, _CHUNK)])
            pltpu.sync_copy(msk_v, msk_hbm.at[pl.ds(base, _CHUNK)])
            return carry

        lax.fori_loop(0, steps, step, 0)

    return k


def kernel(vec, distances, filter_indices):
    E_in = vec.shape[0]
    E_f = filter_indices.shape[0]
    p0, p1, p2, dist_f, switch, msk = _make_kernel(E_in, E_f)(
        vec[:, 0], vec[:, 1], vec[:, 2], distances, filter_indices)
    vec_f = jnp.stack([p0, p1, p2], axis=1)
    return vec_f, dist_f, switch, msk.astype(jnp.bool_)


# speedup vs baseline: 1.0464x; 1.0139x over previous
import functools

import jax
import jax.numpy as jnp
from jax import lax
from jax.experimental import pallas as pl
from jax.experimental.pallas import tpu as pltpu
from jax.experimental.pallas import tpu_sc as plsc

_CUTOFF = 5.0
_L = 16
_NC = 2
_NS = 16
_NW = _NC * _NS
_CHUNK = 800

_PI = 3.141592653589793
_A0 = 0.5
_A1 = -0.5 / 6.0
_A2 = 0.5 / 120.0
_A3 = -0.5 / 5040.0
_A4 = 0.5 / 362880.0


@functools.lru_cache(maxsize=None)
def _make_kernel(E_in: int, E_f: int):
    per_w = E_f // _NW
    steps = per_w // _CHUNK
    assert per_w * _NW == E_f and steps * _CHUNK == per_w

    mesh = plsc.VectorSubcoreMesh(
        core_axis_name="c", subcore_axis_name="s",
        num_cores=_NC, num_subcores=_NS)

    @functools.partial(
        pl.kernel,
        out_type=(
            jax.ShapeDtypeStruct((E_f,), jnp.float32),
            jax.ShapeDtypeStruct((E_f,), jnp.float32),
            jax.ShapeDtypeStruct((E_f,), jnp.float32),
            jax.ShapeDtypeStruct((E_f,), jnp.float32),
            jax.ShapeDtypeStruct((E_f,), jnp.float32),
            jax.ShapeDtypeStruct((E_f,), jnp.int32),
        ),
        mesh=mesh,
        scratch_types=[
            pltpu.VMEM((_CHUNK,), jnp.int32),
            pltpu.VMEM((_CHUNK,), jnp.int32),
            pltpu.VMEM((_CHUNK,), jnp.float32),
            pltpu.VMEM((_CHUNK,), jnp.float32),
            pltpu.VMEM((_CHUNK,), jnp.float32),
            pltpu.VMEM((_CHUNK,), jnp.float32),
            pltpu.VMEM((_CHUNK,), jnp.float32),
            pltpu.VMEM((_CHUNK,), jnp.int32),
            pltpu.SemaphoreType.DMA,
        ],
    )
    def k(t0_hbm, t1_hbm, t2_hbm, dist_hbm, idx_hbm,
          p0_hbm, p1_hbm, p2_hbm, distf_hbm, sw_hbm, msk_hbm,
          idx_raw, idx_c, g0, g1, g2, gd, sw_v, msk_v, sem):
        wid = lax.axis_index("s") * _NC + lax.axis_index("c")
        base0 = wid * per_w

        def step(s, carry):
            base = base0 + s * _CHUNK
            pltpu.sync_copy(idx_hbm.at[pl.ds(base, _CHUNK)], idx_raw)

            def clamp(t, c):
                off = t * _L
                iv = idx_raw[pl.ds(off, _L)]
                idx_c[pl.ds(off, _L)] = jnp.minimum(iv, E_in - 1)
                return c
            lax.fori_loop(0, _CHUNK // _L, clamp, 0)

            c0 = pltpu.async_copy(t0_hbm.at[idx_c], g0, sem)
            c1 = pltpu.async_copy(t1_hbm.at[idx_c], g1, sem)
            c2 = pltpu.async_copy(t2_hbm.at[idx_c], g2, sem)
            cd = pltpu.async_copy(dist_hbm.at[idx_c], gd, sem)
            c0.wait()
            c1.wait()
            c2.wait()
            cd.wait()

            def compute(t, c):
                off = t * _L
                iv = idx_raw[pl.ds(off, _L)]
                valid = iv < E_in
                dg = gd[pl.ds(off, _L)]
                df = jnp.where(valid, dg, _CUTOFF)
                mask = df < _CUTOFF
                u = (df * (1.0 / _CUTOFF) - 0.5) * _PI
                u2 = u * u
                p = _A4 * u2 + _A3
                p = p * u2 + _A2
                p = p * u2 + _A1
                p = p * u2 + _A0
                sw = jnp.where(mask, 0.5 - u * p, 0.0)
                gd[pl.ds(off, _L)] = df
                sw_v[pl.ds(off, _L)] = sw
                msk_v[pl.ds(off, _L)] = jnp.where(mask, 1, 0)
                g0[pl.ds(off, _L)] = jnp.where(valid, g0[pl.ds(off, _L)], _CUTOFF)
                g1[pl.ds(off, _L)] = jnp.where(valid, g1[pl.ds(off, _L)], _CUTOFF)
                g2[pl.ds(off, _L)] = jnp.where(valid, g2[pl.ds(off, _L)], _CUTOFF)
                return c
            lax.fori_loop(0, _CHUNK // _L, compute, 0)

            pltpu.sync_copy(g0, p0_hbm.at[pl.ds(base, _CHUNK)])
            pltpu.sync_copy(g1, p1_hbm.at[pl.ds(base, _CHUNK)])
            pltpu.sync_copy(g2, p2_hbm.at[pl.ds(base, _CHUNK)])
            pltpu.sync_copy(gd, distf_hbm.at[pl.ds(base, _CHUNK)])
            pltpu.sync_copy(sw_v, sw_hbm.at[pl.ds(base, _CHUNK)])
            pltpu.sync_copy(msk_v, msk_hbm.at[pl.ds(base, _CHUNK)])
            return carry

        lax.fori_loop(0, steps, step, 0)

    return k


def kernel(vec, distances, filter_indices):
    E_in = vec.shape[0]
    E_f = filter_indices.shape[0]
    p0, p1, p2, dist_f, switch, msk = _make_kernel(E_in, E_f)(
        vec[:, 0], vec[:, 1], vec[:, 2], distances, filter_indices)
    vec_f = jnp.stack([p0, p1, p2], axis=1)
    return vec_f, dist_f, switch, msk.astype(jnp.bool_)
